# Initial kernel scaffold; baseline (speedup 1.0000x reference)
import jax
import jax.numpy as jnp
from jax import lax
from jax.experimental import pallas as pl
from jax.experimental.pallas import tpu as pltpu
from jax.experimental.pallas import tpu_sc as plsc

N_USERS = 25000
N_NODES = 50000
D = 64
HALF = 25000
ACC_H = 25088
DUMP = ACC_H - 1
E = 800000
K = 128
CT = 392
EP = 16 * CT * K
HALF_CT = CT // 2
ECH = 100
NCH = HALF // ECH
TPS = ACC_H // 16

f32 = jnp.float32
i32 = jnp.int32


def _rsqrt16(x):
    i = plsc.bitcast(x, i32)
    i = jnp.int32(0x5F3759DF) - (i >> 1)
    y = plsc.bitcast(i, f32)
    for _ in range(3):
        y = y * (1.5 - 0.5 * x * y * y)
    return y


def _localize(cbuf, base):
    def tb(j, c):
        for g in range(8):
            v = cbuf[j, pl.ds(g * 16, 16)]
            l = v - base
            ok = (l >= 0) & (l < HALF)
            cbuf[j, pl.ds(g * 16, 16)] = jnp.where(ok, l, DUMP)
        return c
    lax.fori_loop(0, HALF_CT, tb, 0)


def _zero_acc_slice(zbuf, acc, sid):
    r0 = sid * TPS
    for kk in range(12):
        pltpu.sync_copy(zbuf, acc.at[pl.ds(r0 + kk * K, K)])
    pltpu.sync_copy(zbuf.at[pl.ds(0, 32)], acc.at[pl.ds(r0 + 12 * K, 32)])


def _fill(buf, rows, width, val):
    v = jnp.full((16,), val, f32)
    def fb(r, c):
        for g in range(width // 16):
            buf[r, pl.ds(g * 16, 16)] = v
        return c
    lax.fori_loop(0, rows, fb, 0)


def _deg_body(col3, x0, dinv_out, z_out, cbuf, obuf, acc16, sbuf16, xbuf,
              dbuf, zbuf):
    cid = lax.axis_index("c")
    sid = lax.axis_index("s")
    base = cid * HALF
    _fill(obuf, K, 16, 0.0)
    _zero_acc_slice(obuf, acc16, sid)
    _fill(obuf, K, 16, 1.0)
    plsc.subcore_barrier()
    for p in range(2):
        pltpu.sync_copy(col3.at[sid, pl.ds(p * HALF_CT, HALF_CT)], cbuf)
        _localize(cbuf, base)
        def sb(j, c):
            pltpu.sync_copy(obuf, acc16.at[cbuf.at[j]], add=True)
            return c
        lax.fori_loop(0, HALF_CT, sb, 0)
    plsc.subcore_barrier()

    def ep(k, c):
        j = sid + k * 16
        @pl.when(j < NCH)
        def _():
            lrow = j * ECH
            grow = base + lrow
            pltpu.sync_copy(acc16.at[pl.ds(lrow, ECH)], sbuf16)
            pltpu.sync_copy(x0.at[pl.ds(grow, ECH)], xbuf)
            def rw(r, c2):
                dv = sbuf16[r, pl.ds(0, 16)]
                y = _rsqrt16(dv)
                dinv = jnp.where(dv >= 0.5, y, jnp.zeros((16,), f32))
                for g in range(4):
                    dbuf[r, pl.ds(g * 16, 16)] = dinv
                    zbuf[r, pl.ds(g * 16, 16)] = (
                        xbuf[r, pl.ds(g * 16, 16)] * dinv)
                return c2
            lax.fori_loop(0, ECH, rw, 0)
            pltpu.sync_copy(dbuf, dinv_out.at[pl.ds(grow, ECH)])
            pltpu.sync_copy(zbuf, z_out.at[pl.ds(grow, ECH)])
        return c
    lax.fori_loop(0, 16, ep, 0)


def _layer_common(last, row3, col3, z_in, dinv, prev, outs, scratch):
    if last:
        (acc_out,) = outs
        z_next = None
    else:
        acc_out, z_next = outs
    rbuf, cbuf, gbuf, acc, sbuf, dbuf, pbuf = scratch
    cid = lax.axis_index("c")
    sid = lax.axis_index("s")
    base = cid * HALF
    _fill(gbuf, K, D, 0.0)
    _zero_acc_slice(gbuf, acc, sid)
    plsc.subcore_barrier()
    for p in range(2):
        pltpu.sync_copy(row3.at[sid, pl.ds(p * HALF_CT, HALF_CT)], rbuf)
        pltpu.sync_copy(col3.at[sid, pl.ds(p * HALF_CT, HALF_CT)], cbuf)
        _localize(cbuf, base)
        def sb(j, c):
            pltpu.sync_copy(z_in.at[rbuf.at[j]], gbuf)
            pltpu.sync_copy(gbuf, acc.at[cbuf.at[j]], add=True)
            return c
        lax.fori_loop(0, HALF_CT, sb, 0)
    plsc.subcore_barrier()

    def ep(k, c):
        j = sid + k * 16
        @pl.when(j < NCH)
        def _():
            lrow = j * ECH
            grow = base + lrow
            pltpu.sync_copy(acc.at[pl.ds(lrow, ECH)], sbuf)
            pltpu.sync_copy(dinv.at[pl.ds(grow, ECH)], dbuf)
            pltpu.sync_copy(prev.at[pl.ds(grow, ECH)], pbuf)
            def rw(r, c2):
                for g in range(4):
                    s = sbuf[r, pl.ds(g * 16, 16)]
                    d = dbuf[r, pl.ds(g * 16, 16)]
                    y = s * d
                    a = pbuf[r, pl.ds(g * 16, 16)] + y
                    if last:
                        a = a * 0.25
                    pbuf[r, pl.ds(g * 16, 16)] = a
                    if not last:
                        sbuf[r, pl.ds(g * 16, 16)] = y * d
                return c2
            lax.fori_loop(0, ECH, rw, 0)
            pltpu.sync_copy(pbuf, acc_out.at[pl.ds(grow, ECH)])
            if not last:
                pltpu.sync_copy(sbuf, z_next.at[pl.ds(grow, ECH)])
        return c
    lax.fori_loop(0, 16, ep, 0)


def _layer_body(row3, col3, z_in, dinv, prev, acc_out, z_next, rbuf, cbuf,
                gbuf, acc, sbuf, dbuf, pbuf):
    _layer_common(False, row3, col3, z_in, dinv, prev, (acc_out, z_next),
                  (rbuf, cbuf, gbuf, acc, sbuf, dbuf, pbuf))


def _layer_body_last(row3, col3, z_in, dinv, prev, acc_out, rbuf, cbuf,
                     gbuf, acc, sbuf, dbuf, pbuf):
    _layer_common(True, row3, col3, z_in, dinv, prev, (acc_out,),
                  (rbuf, cbuf, gbuf, acc, sbuf, dbuf, pbuf))


def kernel(edge_index, user_emb, item_emb):
    x0 = jnp.concatenate([user_emb, item_emb], axis=0)
    row = edge_index[0].astype(i32)
    col = edge_index[1].astype(i32)
    pad = EP - E
    rowp = jnp.concatenate([row, jnp.zeros((pad,), i32)]).reshape(16, CT, K)
    colp = jnp.concatenate(
        [col, jnp.full((pad,), N_NODES, i32)]).reshape(16, CT, K)

    sds = jax.ShapeDtypeStruct
    mesh = plsc.VectorSubcoreMesh(core_axis_name="c", subcore_axis_name="s")

    deg_k = pl.kernel(
        _deg_body,
        out_type=(sds((N_NODES, D), f32), sds((N_NODES, D), f32)),
        mesh=mesh,
        scratch_types=[
            pltpu.VMEM((HALF_CT, K), i32),
            pltpu.VMEM((K, 16), f32),
            pltpu.VMEM_SHARED((ACC_H, 16), f32),
            pltpu.VMEM((ECH, 16), f32),
            pltpu.VMEM((ECH, D), f32),
            pltpu.VMEM((ECH, D), f32),
            pltpu.VMEM((ECH, D), f32),
        ],
    )
    dinv, z1 = deg_k(colp, x0)

    layer_scratch = [
        pltpu.VMEM((HALF_CT, K), i32),
        pltpu.VMEM((HALF_CT, K), i32),
        pltpu.VMEM((K, D), f32),
        pltpu.VMEM_SHARED((ACC_H, D), f32),
        pltpu.VMEM((ECH, D), f32),
        pltpu.VMEM((ECH, D), f32),
        pltpu.VMEM((ECH, D), f32),
    ]
    layer_k = pl.kernel(
        _layer_body,
        out_type=(sds((N_NODES, D), f32), sds((N_NODES, D), f32)),
        mesh=mesh,
        scratch_types=layer_scratch,
    )
    layer_k_last = pl.kernel(
        _layer_body_last,
        out_type=(sds((N_NODES, D), f32),),
        mesh=mesh,
        scratch_types=layer_scratch,
    )

    acc1, z2 = layer_k(rowp, colp, z1, dinv, x0)
    acc2, z3 = layer_k(rowp, colp, z2, dinv, acc1)
    (acc3,) = layer_k_last(rowp, colp, z3, dinv, acc2)

    return (acc3[:N_USERS], user_emb, acc3[N_USERS:], item_emb)


# speedup vs baseline: 7.1911x; 7.1911x over previous
import jax
import jax.numpy as jnp
from jax import lax
from jax.experimental import pallas as pl
from jax.experimental.pallas import tpu as pltpu
from jax.experimental.pallas import tpu_sc as plsc

N_USERS = 25000
N_NODES = 50000
D = 64
HALF = 25000
ACC_H = 25088
DUMP = ACC_H - 1
E = 800000
K = 128
NP = 8
PC = 49
EP = 16 * NP * PC * K
ECH = 40
NCH = HALF // ECH
TPS = ACC_H // 16

f32 = jnp.float32
i32 = jnp.int32


def _rsqrt16(x):
    i = lax.bitcast_convert_type(x, i32)
    i = jnp.int32(0x5F3759DF) - (i >> 1)
    y = lax.bitcast_convert_type(i, f32)
    for _ in range(3):
        y = y * (1.5 - 0.5 * x * y * y)
    return y


def _layer_body(row4, col4, z_in, dinv, prev, mul, sel, acc_out, z_next,
                rbuf, cbuf, gbuf, acc, mbuf, slbuf):
    cid = lax.axis_index("c")
    sid = lax.axis_index("s")
    base = cid * HALF
    zero16 = jnp.zeros((16,), f32)

    def gz(r, c):
        for g in range(4):
            gbuf[r, pl.ds(g * 16, 16)] = zero16
        return c
    lax.fori_loop(0, K, gz, 0)
    r0 = sid * TPS
    for kk in range(12):
        pltpu.sync_copy(gbuf, acc.at[pl.ds(r0 + kk * K, K)])
    pltpu.sync_copy(gbuf.at[pl.ds(0, 32)], acc.at[pl.ds(r0 + 12 * K, 32)])
    pltpu.sync_copy(mul, mbuf)
    pltpu.sync_copy(sel, slbuf)
    plsc.subcore_barrier()

    for p in range(NP):
        pltpu.sync_copy(row4.at[sid, p], rbuf)
        pltpu.sync_copy(col4.at[sid, p], cbuf)
        def tb(j, c):
            for g in range(8):
                v = cbuf[j, pl.ds(g * 16, 16)]
                l = v - base
                ok = (l >= 0) & (l < HALF)
                cbuf[j, pl.ds(g * 16, 16)] = jnp.where(ok, l, DUMP)
            return c
        lax.fori_loop(0, PC, tb, 0)
        def sb(j, c):
            pltpu.sync_copy(z_in.at[rbuf.at[j]], gbuf)
            pltpu.sync_copy(gbuf, acc.at[cbuf.at[j]], add=True)
            return c
        lax.fori_loop(0, PC, sb, 0)
    plsc.subcore_barrier()

    mv = mbuf[pl.ds(0, 16)]
    selv = slbuf[pl.ds(0, 16)]
    def ep(k, c):
        j = sid + k * 16
        @pl.when(j < NCH)
        def _():
            lrow = j * ECH
            grow = base + lrow
            pltpu.sync_copy(acc.at[pl.ds(lrow, ECH)], gbuf.at[pl.ds(0, ECH)])
            pltpu.sync_copy(dinv.at[pl.ds(grow, ECH)],
                            gbuf.at[pl.ds(ECH, ECH)])
            pltpu.sync_copy(prev.at[pl.ds(grow, ECH)],
                            gbuf.at[pl.ds(2 * ECH, ECH)])
            def rw(r, c2):
                for g in range(4):
                    s = gbuf[r, pl.ds(g * 16, 16)]
                    d = gbuf[ECH + r, pl.ds(g * 16, 16)]
                    pv = gbuf[2 * ECH + r, pl.ds(g * 16, 16)]
                    y = s * d
                    dv = jnp.where(s >= 0.5, _rsqrt16(s), zero16)
                    use_dv = selv > 0.5
                    ao = jnp.where(use_dv, dv, (pv + y) * mv)
                    zn = jnp.where(use_dv, dv * pv, y * d)
                    gbuf[ECH + r, pl.ds(g * 16, 16)] = ao
                    gbuf[2 * ECH + r, pl.ds(g * 16, 16)] = zn
                return c2
            lax.fori_loop(0, ECH, rw, 0)
            pltpu.sync_copy(gbuf.at[pl.ds(ECH, ECH)],
                            acc_out.at[pl.ds(grow, ECH)])
            pltpu.sync_copy(gbuf.at[pl.ds(2 * ECH, ECH)],
                            z_next.at[pl.ds(grow, ECH)])
        return c
    lax.fori_loop(0, (NCH + 15) // 16, ep, 0)


def kernel(edge_index, user_emb, item_emb):
    x0 = jnp.concatenate([user_emb, item_emb], axis=0)
    row = edge_index[0].astype(i32)
    col = edge_index[1].astype(i32)
    pad = EP - E
    rowp = jnp.concatenate(
        [row, jnp.zeros((pad,), i32)]).reshape(16, NP, PC, K)
    colp = jnp.concatenate(
        [col, jnp.full((pad,), N_NODES, i32)]).reshape(16, NP, PC, K)
    one16 = jnp.ones((16,), f32)
    quarter16 = jnp.full((16,), 0.25, f32)
    zero16c = jnp.zeros((16,), f32)
    ones_t = jnp.ones((N_NODES, D), f32)

    sds = jax.ShapeDtypeStruct
    mesh = plsc.VectorSubcoreMesh(core_axis_name="c", subcore_axis_name="s")

    layer_k = pl.kernel(
        _layer_body,
        out_type=(sds((N_NODES, D), f32), sds((N_NODES, D), f32)),
        mesh=mesh,
        compiler_params=pltpu.CompilerParams(use_tc_tiling_on_sc=False),
        scratch_types=[
            pltpu.VMEM((PC, K), i32),
            pltpu.VMEM((PC, K), i32),
            pltpu.VMEM((K, D), f32),
            pltpu.VMEM_SHARED((ACC_H, D), f32),
            pltpu.VMEM((16,), f32),
            pltpu.VMEM((16,), f32),
        ],
    )

    dinv, z1 = layer_k(rowp, colp, ones_t, ones_t, x0, one16, one16)
    acc1, z2 = layer_k(rowp, colp, z1, dinv, x0, one16, zero16c)
    acc2, z3 = layer_k(rowp, colp, z2, dinv, acc1, one16, zero16c)
    acc3, _ = layer_k(rowp, colp, z3, dinv, acc2, quarter16, zero16c)

    return (acc3[:N_USERS], user_emb, acc3[N_USERS:], item_emb)
